# baseline (speedup 1.0000x reference)
import functools

import jax
import jax.numpy as jnp
from jax import lax
from jax.experimental import pallas as pl
from jax.experimental.pallas import tpu as pltpu
from jax.experimental.pallas import tpu_sc as plsc

N_BATCH = 16
N_CODES = 1024
DIM = 64
N_PIX = 1024
BLK_P = 512
C_HALF = 32
LANES = 16


CHUNK = 256


def _dist_body(x_ref, e_ref, ee2_ref, xs2_ref, idx_ref, loss_ref, et_ref):
    X = x_ref[0].reshape(DIM, BLK_P)
    xs2 = xs2_ref[0]
    m = None
    idx = None
    for c in range(N_CODES // CHUNK):
        Ec = e_ref[pl.ds(c * CHUNK, CHUNK), :]
        Mc = jax.lax.dot_general(Ec, X, (((1,), (0,)), ((), ())),
                                 preferred_element_type=jnp.float32)
        dc = (xs2 + ee2_ref[pl.ds(c * CHUNK, CHUNK), :]) - 2.0 * Mc
        mc = jnp.min(dc, axis=0, keepdims=True)
        iota = jax.lax.broadcasted_iota(jnp.int32, dc.shape, 0)
        gc = jnp.min(jnp.where(dc == mc, iota, N_CODES), axis=0,
                     keepdims=True) + jnp.int32(c * CHUNK)
        if m is None:
            m, idx = mc, gc
        else:
            idx = jnp.where(mc < m, gc, idx)
            m = jnp.minimum(m, mc)
    idx_ref[0] = idx
    part = jnp.sum(m, axis=1, keepdims=True)

    @pl.when((pl.program_id(0) == 0) & (pl.program_id(1) == 0))
    def _init():
        loss_ref[...] = jnp.zeros_like(loss_ref)
        et_ref[...] = e_ref[...].T

    loss_ref[...] += part


_sc_mesh = plsc.VectorSubcoreMesh(core_axis_name="c", subcore_axis_name="s")


@functools.partial(
    pl.kernel,
    out_type=jax.ShapeDtypeStruct((N_BATCH, DIM, N_PIX), jnp.float32),
    mesh=_sc_mesh,
    scratch_types=[
        pltpu.VMEM((N_PIX,), jnp.int32),
        pltpu.VMEM((C_HALF * N_PIX,), jnp.float32),
        pltpu.VMEM((C_HALF, N_PIX), jnp.float32),
    ],
    compiler_params=pltpu.CompilerParams(needs_layout_passes=False),
)
def _sc_gather(et_hbm, idx_hbm, out_hbm, idx_v, et_v, out_v):
    wid = lax.axis_index("s") * 2 + lax.axis_index("c")
    n = wid // 2
    c0 = (wid % 2) * C_HALF
    pltpu.sync_copy(et_hbm.at[pl.ds(c0 * N_PIX, C_HALF * N_PIX)], et_v)
    pltpu.sync_copy(idx_hbm.at[pl.ds(n * N_PIX, N_PIX)], idx_v)

    def body(g, carry):
        base = pl.multiple_of(g * LANES, LANES)
        idxg = idx_v[pl.ds(base, LANES)]
        for c in range(C_HALF):
            out_v[c, pl.ds(base, LANES)] = plsc.load_gather(
                et_v, [idxg + jnp.int32(c * N_PIX)])
        return carry

    lax.fori_loop(0, N_PIX // LANES, body, 0)
    pltpu.sync_copy(out_v, out_hbm.at[n, pl.ds(c0, C_HALF), :])


@jax.jit
def kernel(inputs, embedding):
    ee2 = jnp.sum(embedding ** 2, axis=1).reshape(N_CODES, 1)
    xs2 = jnp.sum(jnp.transpose(inputs, (0, 2, 3, 1)).reshape(-1, DIM) ** 2,
                  axis=1).reshape(N_BATCH, 1, N_PIX)

    grid = (N_BATCH, N_PIX // BLK_P)
    idx3, loss_sum, et = pl.pallas_call(
        _dist_body,
        grid=grid,
        in_specs=[
            pl.BlockSpec((1, DIM, BLK_P // 32, 32), lambda n, b: (n, 0, b, 0)),
            pl.BlockSpec((N_CODES, DIM), lambda n, b: (0, 0)),
            pl.BlockSpec((N_CODES, 1), lambda n, b: (0, 0)),
            pl.BlockSpec((1, 1, BLK_P), lambda n, b: (n, 0, b)),
        ],
        out_specs=[
            pl.BlockSpec((1, 1, BLK_P), lambda n, b: (n, 0, b)),
            pl.BlockSpec((1, 1), lambda n, b: (0, 0)),
            pl.BlockSpec((DIM, N_CODES), lambda n, b: (0, 0)),
        ],
        out_shape=[
            jax.ShapeDtypeStruct((N_BATCH, 1, N_PIX), jnp.int32),
            jax.ShapeDtypeStruct((1, 1), jnp.float32),
            jax.ShapeDtypeStruct((DIM, N_CODES), jnp.float32),
        ],
    )(inputs, embedding, ee2, xs2)

    q2 = _sc_gather(et.reshape(-1), idx3.reshape(N_BATCH * N_PIX))

    n_elems = N_BATCH * DIM * N_PIX
    loss = (1.25 / n_elems) * loss_sum[0, 0]
    return loss, q2.reshape(inputs.shape)


# speedup vs baseline: 1.1702x; 1.1702x over previous
import functools

import jax
import jax.numpy as jnp
from jax import lax
from jax.experimental import pallas as pl
from jax.experimental.pallas import tpu as pltpu
from jax.experimental.pallas import tpu_sc as plsc

N_BATCH = 16
N_CODES = 1024
DIM = 64
N_PIX = 1024
BLK_P = 512
NB = 8
C_Q = 16
LANES = 16
CHUNK = 256


def _argmin_chunks(X, e_ref, ee2_ref, xs2):
    m = None
    idx = None
    for c in range(N_CODES // CHUNK):
        Ec = e_ref[pl.ds(c * CHUNK, CHUNK), :]
        Mc = jax.lax.dot_general(Ec, X, (((1,), (0,)), ((), ())),
                                 preferred_element_type=jnp.float32)
        dc = (xs2 + ee2_ref[pl.ds(c * CHUNK, CHUNK), :]) - 2.0 * Mc
        mc = jnp.min(dc, axis=0, keepdims=True)
        iota = jax.lax.broadcasted_iota(jnp.int32, dc.shape, 0)
        gc = jnp.min(jnp.where(dc == mc, iota, N_CODES), axis=0,
                     keepdims=True) + jnp.int32(c * CHUNK)
        if m is None:
            m, idx = mc, gc
        else:
            idx = jnp.where(mc < m, gc, idx)
            m = jnp.minimum(m, mc)
    return m, idx


def _dist_body_et(x_ref, e_ref, ee2_ref, xs2_ref, idx_ref, loss_ref, et_ref):
    m, idx = _argmin_chunks(x_ref[0], e_ref, ee2_ref, xs2_ref[0])
    idx_ref[0] = idx
    part = jnp.sum(m, axis=1, keepdims=True)

    @pl.when((pl.program_id(0) == 0) & (pl.program_id(1) == 0))
    def _init():
        loss_ref[...] = jnp.zeros_like(loss_ref)
        et_ref[...] = e_ref[...].T

    loss_ref[...] += part


def _dist_body(x_ref, e_ref, ee2_ref, xs2_ref, idx_ref, loss_ref):
    m, idx = _argmin_chunks(x_ref[0], e_ref, ee2_ref, xs2_ref[0])
    idx_ref[0] = idx
    part = jnp.sum(m, axis=1, keepdims=True)

    @pl.when((pl.program_id(0) == 0) & (pl.program_id(1) == 0))
    def _init():
        loss_ref[...] = jnp.zeros_like(loss_ref)

    loss_ref[...] += part


def _dist_call(x3, embedding, ee2, xs2, with_et):
    grid = (NB, N_PIX // BLK_P)
    in_specs = [
        pl.BlockSpec((1, DIM, BLK_P), lambda n, b: (n, 0, b)),
        pl.BlockSpec((N_CODES, DIM), lambda n, b: (0, 0)),
        pl.BlockSpec((N_CODES, 1), lambda n, b: (0, 0)),
        pl.BlockSpec((1, 1, BLK_P), lambda n, b: (n, 0, b)),
    ]
    out_specs = [
        pl.BlockSpec((1, 1, BLK_P), lambda n, b: (n, 0, b)),
        pl.BlockSpec((1, 1), lambda n, b: (0, 0)),
    ]
    out_shape = [
        jax.ShapeDtypeStruct((NB, 1, N_PIX), jnp.int32),
        jax.ShapeDtypeStruct((1, 1), jnp.float32),
    ]
    if with_et:
        out_specs.append(pl.BlockSpec((DIM, N_CODES), lambda n, b: (0, 0)))
        out_shape.append(jax.ShapeDtypeStruct((DIM, N_CODES), jnp.float32))
    return pl.pallas_call(
        _dist_body_et if with_et else _dist_body,
        grid=grid, in_specs=in_specs, out_specs=out_specs, out_shape=out_shape,
    )(x3, embedding, ee2, xs2)


_sc_mesh = plsc.VectorSubcoreMesh(core_axis_name="c", subcore_axis_name="s")


@functools.partial(
    pl.kernel,
    out_type=jax.ShapeDtypeStruct((NB, DIM, N_PIX), jnp.float32),
    mesh=_sc_mesh,
    scratch_types=[
        pltpu.VMEM((N_PIX,), jnp.int32),
        pltpu.VMEM((C_Q * N_CODES,), jnp.float32),
        pltpu.VMEM((C_Q, N_PIX), jnp.float32),
    ],
    compiler_params=pltpu.CompilerParams(needs_layout_passes=False),
)
def _sc_gather(et_hbm, idx_hbm, out_hbm, idx_v, et_v, out_v):
    wid = lax.axis_index("s") * 2 + lax.axis_index("c")
    n = wid // 4
    c0 = (wid % 4) * C_Q
    pltpu.sync_copy(et_hbm.at[pl.ds(c0 * N_CODES, C_Q * N_CODES)], et_v)
    pltpu.sync_copy(idx_hbm.at[pl.ds(n * N_PIX, N_PIX)], idx_v)

    def body(g, carry):
        base = pl.multiple_of(g * LANES, LANES)
        idxg = idx_v[pl.ds(base, LANES)]
        for c in range(C_Q):
            out_v[c, pl.ds(base, LANES)] = plsc.load_gather(
                et_v, [idxg + jnp.int32(c * N_CODES)])
        return carry

    lax.fori_loop(0, N_PIX // LANES, body, 0)
    pltpu.sync_copy(out_v, out_hbm.at[n, pl.ds(c0, C_Q), :])


@jax.jit
def kernel(inputs, embedding):
    x3 = inputs.reshape(N_BATCH, DIM, N_PIX)
    ee2 = jnp.sum(embedding ** 2, axis=1).reshape(N_CODES, 1)
    xs2 = jnp.sum(jnp.transpose(inputs, (0, 2, 3, 1)).reshape(-1, DIM) ** 2,
                  axis=1).reshape(N_BATCH, 1, N_PIX)

    idx_a, loss_a, et = _dist_call(x3[:NB], embedding, ee2, xs2[:NB], True)
    idx_b, loss_b = _dist_call(x3[NB:], embedding, ee2, xs2[NB:], False)
    etf = et.reshape(-1)
    q_a = _sc_gather(etf, idx_a.reshape(NB * N_PIX))
    q_b = _sc_gather(etf, idx_b.reshape(NB * N_PIX))

    n_elems = N_BATCH * DIM * N_PIX
    loss = (1.25 / n_elems) * (loss_a[0, 0] + loss_b[0, 0])
    q = jnp.concatenate([q_a, q_b], axis=0)
    return loss, q.reshape(inputs.shape)


# speedup vs baseline: 1.2386x; 1.0584x over previous
import functools

import jax
import jax.numpy as jnp
from jax import lax
from jax.experimental import pallas as pl
from jax.experimental.pallas import tpu as pltpu
from jax.experimental.pallas import tpu_sc as plsc

N_BATCH = 16
N_CODES = 1024
DIM = 64
N_PIX = 1024
BLK_P = 512
NB = 8
C_Q = 16
LANES = 16
CHUNK = 256


def _dist_body(x_ref, e_ref, ee2_ref, xs2_ref, idx_ref, loss_ref):
    X = x_ref[0]
    xs2 = xs2_ref[0]
    m = None
    idx = None
    for c in range(N_CODES // CHUNK):
        Ec = e_ref[pl.ds(c * CHUNK, CHUNK), :]
        Mc = jax.lax.dot_general(Ec, X, (((1,), (0,)), ((), ())),
                                 preferred_element_type=jnp.float32)
        dc = (xs2 + ee2_ref[pl.ds(c * CHUNK, CHUNK), :]) - 2.0 * Mc
        mc = jnp.min(dc, axis=0, keepdims=True)
        iota = jax.lax.broadcasted_iota(jnp.int32, dc.shape, 0)
        gc = jnp.min(jnp.where(dc == mc, iota, N_CODES), axis=0,
                     keepdims=True) + jnp.int32(c * CHUNK)
        if m is None:
            m, idx = mc, gc
        else:
            idx = jnp.where(mc < m, gc, idx)
            m = jnp.minimum(m, mc)
    idx_ref[0] = idx
    part = jnp.sum(m, axis=1, keepdims=True)

    @pl.when((pl.program_id(0) == 0) & (pl.program_id(1) == 0))
    def _init():
        loss_ref[...] = jnp.zeros_like(loss_ref)

    loss_ref[...] += part


def _dist_body_et(x_ref, e_ref, ee2_ref, xs2_ref, idx_ref, loss_ref, et_ref):
    _dist_body(x_ref, e_ref, ee2_ref, xs2_ref, idx_ref, loss_ref)

    @pl.when((pl.program_id(0) == 0) & (pl.program_id(1) == 0))
    def _init():
        et_ref[...] = e_ref[...].T


def _dist_call(x3, embedding, ee2, xs2, base, with_et):
    grid = (NB, N_PIX // BLK_P)
    out_specs = [
        pl.BlockSpec((1, 1, BLK_P), lambda n, b: (n, 0, b)),
        pl.BlockSpec((1, 1), lambda n, b: (0, 0)),
    ]
    out_shape = [
        jax.ShapeDtypeStruct((NB, 1, N_PIX), jnp.int32),
        jax.ShapeDtypeStruct((1, 1), jnp.float32),
    ]
    if with_et:
        out_specs.append(pl.BlockSpec((DIM, N_CODES), lambda n, b: (0, 0)))
        out_shape.append(jax.ShapeDtypeStruct((DIM, N_CODES), jnp.float32))
    return pl.pallas_call(
        _dist_body_et if with_et else _dist_body,
        grid=grid,
        in_specs=[
            pl.BlockSpec((1, DIM, BLK_P), lambda n, b: (n + base, 0, b)),
            pl.BlockSpec((N_CODES, DIM), lambda n, b: (0, 0)),
            pl.BlockSpec((N_CODES, 1), lambda n, b: (0, 0)),
            pl.BlockSpec((1, 1, BLK_P), lambda n, b: (n + base, 0, b)),
        ],
        out_specs=out_specs,
        out_shape=out_shape,
    )(x3, embedding, ee2, xs2)


_sc_mesh = plsc.VectorSubcoreMesh(core_axis_name="c", subcore_axis_name="s")


@functools.partial(
    pl.kernel,
    out_type=jax.ShapeDtypeStruct((NB, DIM, N_PIX), jnp.float32),
    mesh=_sc_mesh,
    scratch_types=[
        pltpu.VMEM((N_PIX,), jnp.int32),
        pltpu.VMEM((C_Q, N_CODES), jnp.float32),
        pltpu.VMEM((C_Q, N_PIX), jnp.float32),
    ],
    compiler_params=pltpu.CompilerParams(needs_layout_passes=False),
)
def _sc_gather(et_hbm, idx_hbm, out_hbm, idx_v, es_v, out_v):
    wid = lax.axis_index("s") * 2 + lax.axis_index("c")
    n = wid // 4
    c0 = (wid % 4) * C_Q
    pltpu.sync_copy(et_hbm.at[pl.ds(c0, C_Q), :], es_v)
    pltpu.sync_copy(idx_hbm.at[n, 0, :], idx_v)

    def body(g, carry):
        base = pl.multiple_of(g * LANES, LANES)
        idxg = idx_v[pl.ds(base, LANES)]
        for cc in range(C_Q):
            out_v[cc, pl.ds(base, LANES)] = plsc.load_gather(
                es_v, [jnp.full((LANES,), cc, jnp.int32), idxg])
        return carry

    lax.fori_loop(0, N_PIX // LANES, body, 0)
    pltpu.sync_copy(out_v, out_hbm.at[n, pl.ds(c0, C_Q), :])


@jax.jit
def kernel(inputs, embedding):
    x3 = inputs.reshape(N_BATCH, DIM, N_PIX)
    ee2 = jnp.sum(embedding ** 2, axis=1).reshape(N_CODES, 1)
    xs2 = jnp.sum(jnp.transpose(inputs, (0, 2, 3, 1)).reshape(-1, DIM) ** 2,
                  axis=1).reshape(N_BATCH, 1, N_PIX)

    idx_a, loss_a, et = _dist_call(x3, embedding, ee2, xs2, 0, True)
    idx_b, loss_b = _dist_call(x3, embedding, ee2, xs2, NB, False)
    q_a = _sc_gather(et, idx_a)
    q_b = _sc_gather(et, idx_b)

    n_elems = N_BATCH * DIM * N_PIX
    loss = (1.25 / n_elems) * (loss_a[0, 0] + loss_b[0, 0])
    q = jnp.concatenate([q_a, q_b], axis=0)
    return loss, q.reshape(inputs.shape)
